# baseline (speedup 1.0000x reference)
import functools

import jax
import jax.numpy as jnp
from jax.experimental import pallas as pl
from jax.experimental.pallas import tpu as pltpu

T = 2048
D = 1024
H = 2048
O = 1024
E = 8

BT = 256
BH = 512


def _moe_dense_kernel(x_ref, Wg_ref, bg_ref, W1_ref, b1_ref, W2_ref, b2_ref,
                      out_ref, wfull_ref):
    e = pl.program_id(1)
    j = pl.program_id(2)

    @pl.when((e == 0) & (j == 0))
    def _gate():
        logits = jnp.dot(x_ref[...], Wg_ref[...],
                         preferred_element_type=jnp.float32) + bg_ref[...]
        idx = jax.lax.broadcasted_iota(jnp.int32, (BT, E), 1)
        m1 = jnp.max(logits, axis=1, keepdims=True)
        i1 = jnp.min(jnp.where(logits >= m1, idx, E), axis=1, keepdims=True)
        oh1 = idx == i1
        neg = jnp.float32(-jnp.inf)
        rest = jnp.where(oh1, neg, logits)
        m2 = jnp.max(rest, axis=1, keepdims=True)
        i2 = jnp.min(jnp.where(rest >= m2, idx, E), axis=1, keepdims=True)
        oh2 = idx == i2
        r = jnp.exp(m2 - m1)
        w1 = 1.0 / (1.0 + r)
        w2 = r / (1.0 + r)
        wfull_ref[...] = jnp.where(oh1, w1, 0.0) + jnp.where(oh2, w2, 0.0)
        out_ref[...] = jnp.zeros_like(out_ref)

    idx = jax.lax.broadcasted_iota(jnp.int32, (BT, E), 1)
    w_col = jnp.sum(jnp.where(idx == e, wfull_ref[...], 0.0), axis=1,
                    keepdims=True)

    h = jnp.maximum(
        jnp.dot(x_ref[...], W1_ref[0], preferred_element_type=jnp.float32)
        + b1_ref[0], 0.0)
    part = jnp.dot(h, W2_ref[0], preferred_element_type=jnp.float32)
    part = jnp.where(j == 0, part + b2_ref[0], part)
    out_ref[...] += w_col * part


@jax.jit
def kernel(x, Wg, bg, W1, b1, W2, b2):
    grid = (T // BT, E, H // BH)
    return pl.pallas_call(
        _moe_dense_kernel,
        grid=grid,
        in_specs=[
            pl.BlockSpec((BT, D), lambda t, e, j: (t, 0)),
            pl.BlockSpec((D, E), lambda t, e, j: (0, 0)),
            pl.BlockSpec((1, E), lambda t, e, j: (0, 0)),
            pl.BlockSpec((1, D, BH), lambda t, e, j: (e, 0, j)),
            pl.BlockSpec((1, 1, BH), lambda t, e, j: (e, 0, j)),
            pl.BlockSpec((1, BH, O), lambda t, e, j: (e, j, 0)),
            pl.BlockSpec((1, 1, O), lambda t, e, j: (e, 0, 0)),
        ],
        out_specs=pl.BlockSpec((BT, O), lambda t, e, j: (t, 0)),
        out_shape=jax.ShapeDtypeStruct((T, O), jnp.float32),
        scratch_shapes=[pltpu.VMEM((BT, E), jnp.float32)],
    )(x, Wg, bg.reshape(1, E), W1, b1.reshape(E, 1, H), W2,
      b2.reshape(E, 1, O))


# speedup vs baseline: 1.3616x; 1.3616x over previous
import functools

import jax
import jax.numpy as jnp
from jax import lax
from jax.experimental import pallas as pl
from jax.experimental.pallas import tpu as pltpu
from jax.experimental.pallas import tpu_sc as plsc

T = 2048
D = 1024
H = 2048
O = 1024
E = 8
K = 2

BT = 256
NE = T * K
NB = (NE + E * (BT - 1) + BT - 1) // BT
P = NB * BT

NWORKERS = 32


def _gating_body(x_ref, Wg_ref, bg_ref, ti_ref, tw_ref):
    logits = jnp.dot(x_ref[...], Wg_ref[...],
                     preferred_element_type=jnp.float32) + bg_ref[...]
    idx = lax.broadcasted_iota(jnp.int32, (T, E), 1)
    m1 = jnp.max(logits, axis=1, keepdims=True)
    i1 = jnp.min(jnp.where(logits >= m1, idx, E), axis=1, keepdims=True)
    rest = jnp.where(idx == i1, -jnp.inf, logits)
    m2 = jnp.max(rest, axis=1, keepdims=True)
    i2 = jnp.min(jnp.where(rest >= m2, idx, E), axis=1, keepdims=True)
    r = jnp.exp(m2 - m1)
    w1 = 1.0 / (1.0 + r)
    ti_ref[...] = jnp.concatenate([i1, i2], axis=1)
    tw_ref[...] = jnp.concatenate([w1, 1.0 - w1], axis=1)


def _gating(x, Wg, bg):
    return pl.pallas_call(
        _gating_body,
        out_shape=(jax.ShapeDtypeStruct((T, K), jnp.int32),
                   jax.ShapeDtypeStruct((T, K), jnp.float32)),
    )(x, Wg, bg.reshape(1, E))


def _route_meta(top_i, top_w):
    f = top_i.reshape(-1)
    w = top_w.reshape(-1)
    oh = (f[:, None] == jnp.arange(E, dtype=jnp.int32)[None, :])
    ohi = oh.astype(jnp.int32)
    ranks = jnp.cumsum(ohi, axis=0) - ohi
    rank = jnp.sum(ranks * ohi, axis=1)
    counts = jnp.sum(ohi, axis=0)
    nblk = (counts + BT - 1) // BT
    blk_end = jnp.cumsum(nblk)
    blk_start = jnp.concatenate([jnp.zeros((1,), jnp.int32), blk_end[:-1]])
    dest = (blk_start[f] * BT + rank).astype(jnp.int32)

    tok = (jnp.arange(NE, dtype=jnp.int32) // K)
    gather_idx = jnp.zeros((P,), jnp.int32).at[dest].set(tok)
    row_w = jnp.zeros((P,), jnp.float32).at[dest].set(w)

    nb_used = blk_end[E - 1]
    bidx = jnp.arange(NB, dtype=jnp.int32)
    be = jnp.searchsorted(blk_end, bidx, side="right").astype(jnp.int32)
    be_last = be[nb_used - 1]
    block_expert = jnp.where(bidx < nb_used, be, be_last)
    block_valid = (bidx < nb_used).astype(jnp.int32)
    return dest, gather_idx, row_w, block_expert, block_valid


def _make_sc_gather(n_rows, d, chunk):
    per_w = n_rows // NWORKERS
    n_chunks = per_w // chunk
    mesh = plsc.VectorSubcoreMesh(core_axis_name="c", subcore_axis_name="s")

    def body(src_hbm, idx_hbm, out_hbm, idx_v, rows_v, sem):
        wid = lax.axis_index("s") * 2 + lax.axis_index("c")
        base = wid * per_w

        def step(i, carry):
            off = base + i * chunk
            pltpu.sync_copy(idx_hbm.at[pl.ds(off, chunk)], idx_v)
            pltpu.async_copy(src_hbm.at[idx_v], rows_v, sem).wait()
            pltpu.sync_copy(rows_v, out_hbm.at[pl.ds(off, chunk)])
            return carry

        lax.fori_loop(0, n_chunks, step, 0)

    return pl.kernel(
        body,
        out_type=jax.ShapeDtypeStruct((n_rows, d), jnp.float32),
        mesh=mesh,
        scratch_types=[
            pltpu.VMEM((chunk,), jnp.int32),
            pltpu.VMEM((chunk, d), jnp.float32),
            pltpu.SemaphoreType.DMA,
        ],
    )


@functools.cache
def _sc_gather(n_rows, d, chunk):
    return _make_sc_gather(n_rows, d, chunk)


def _sc_gather_x(src, idx):
    return _sc_gather(P, D, 64)(src, idx)


def _sc_gather_y(src, idx):
    return _sc_gather(NE, O, 64)(src, idx)


def _grouped_body(be_ref, valid_ref, xs_ref, W1_ref, b1_ref, W2_ref, b2_ref,
                  rw_ref, out_ref):
    b = pl.program_id(0)

    @pl.when(valid_ref[b] == 1)
    def _():
        h = jnp.maximum(
            jnp.dot(xs_ref[...], W1_ref[0],
                    preferred_element_type=jnp.float32) + b1_ref[0], 0.0)
        y = jnp.dot(h, W2_ref[0], preferred_element_type=jnp.float32)
        out_ref[...] = (y + b2_ref[0]) * rw_ref[...]


def _grouped_mlp(xs, W1, b1, W2, b2, row_w, block_expert, block_valid):
    grid_spec = pltpu.PrefetchScalarGridSpec(
        num_scalar_prefetch=2,
        grid=(NB,),
        in_specs=[
            pl.BlockSpec((BT, D), lambda b, be, v: (b, 0)),
            pl.BlockSpec((1, D, H), lambda b, be, v: (be[b], 0, 0)),
            pl.BlockSpec((1, 1, H), lambda b, be, v: (be[b], 0, 0)),
            pl.BlockSpec((1, H, O), lambda b, be, v: (be[b], 0, 0)),
            pl.BlockSpec((1, 1, O), lambda b, be, v: (be[b], 0, 0)),
            pl.BlockSpec((BT, 1), lambda b, be, v: (b, 0)),
        ],
        out_specs=pl.BlockSpec((BT, O), lambda b, be, v: (b, 0)),
    )
    return pl.pallas_call(
        _grouped_body,
        grid_spec=grid_spec,
        out_shape=jax.ShapeDtypeStruct((P, O), jnp.float32),
    )(block_expert, block_valid, xs, W1, b1.reshape(E, 1, H), W2,
      b2.reshape(E, 1, O), row_w.reshape(P, 1))


def _combine_body(g_ref, out_ref):
    out_ref[...] = g_ref[:, 0, :] + g_ref[:, 1, :]


def _combine(g):
    btc = 512
    return pl.pallas_call(
        _combine_body,
        grid=(T // btc,),
        in_specs=[pl.BlockSpec((btc, K, O), lambda t: (t, 0, 0))],
        out_specs=pl.BlockSpec((btc, O), lambda t: (t, 0)),
        out_shape=jax.ShapeDtypeStruct((T, O), jnp.float32),
    )(g)


@jax.jit
def kernel(x, Wg, bg, W1, b1, W2, b2):
    top_i, top_w = _gating(x, Wg, bg)
    dest, gather_idx, row_w, block_expert, block_valid = _route_meta(
        top_i, top_w)
    xs = _sc_gather_x(x, gather_idx)
    yw = _grouped_mlp(xs, W1, b1, W2, b2, row_w, block_expert, block_valid)
    g = _sc_gather_y(yw, dest)
    return _combine(g.reshape(T, K, O))


# speedup vs baseline: 1.9683x; 1.4455x over previous
import functools

import jax
import jax.numpy as jnp
from jax import lax
from jax.experimental import pallas as pl
from jax.experimental.pallas import tpu as pltpu
from jax.experimental.pallas import tpu_sc as plsc

T = 2048
D = 1024
H = 2048
O = 1024
E = 8
K = 2

BT = 256
NE = T * K
NB = (NE + E * (BT - 1) + BT - 1) // BT
P = NB * BT

NWORKERS = 32


def _gating_body(x_ref, Wg_ref, bg_ref, ti_ref, tw_ref):
    logits = jnp.dot(x_ref[...], Wg_ref[...],
                     preferred_element_type=jnp.float32) + bg_ref[...]
    idx = lax.broadcasted_iota(jnp.int32, (T, E), 1)
    m1 = jnp.max(logits, axis=1, keepdims=True)
    i1 = jnp.min(jnp.where(logits >= m1, idx, E), axis=1, keepdims=True)
    rest = jnp.where(idx == i1, -jnp.inf, logits)
    m2 = jnp.max(rest, axis=1, keepdims=True)
    i2 = jnp.min(jnp.where(rest >= m2, idx, E), axis=1, keepdims=True)
    r = jnp.exp(m2 - m1)
    w1 = 1.0 / (1.0 + r)
    ti_ref[...] = jnp.concatenate([i1, i2], axis=1)
    tw_ref[...] = jnp.concatenate([w1, 1.0 - w1], axis=1)


def _gating(x, Wg, bg):
    return pl.pallas_call(
        _gating_body,
        out_shape=(jax.ShapeDtypeStruct((T, K), jnp.int32),
                   jax.ShapeDtypeStruct((T, K), jnp.float32)),
    )(x, Wg, bg.reshape(1, E))


def _route_meta(top_i, top_w):
    f = top_i.reshape(-1)
    w = top_w.reshape(-1)
    oh = (f[:, None] == jnp.arange(E, dtype=jnp.int32)[None, :])
    ohi = oh.astype(jnp.int32)
    ranks = jnp.cumsum(ohi, axis=0) - ohi
    rank = jnp.sum(ranks * ohi, axis=1)
    counts = jnp.sum(ohi, axis=0)
    nblk = (counts + BT - 1) // BT
    blk_end = jnp.cumsum(nblk)
    blk_start = jnp.concatenate([jnp.zeros((1,), jnp.int32), blk_end[:-1]])
    dest = (blk_start[f] * BT + rank).astype(jnp.int32)

    tok = (jnp.arange(NE, dtype=jnp.int32) // K)
    gather_idx = (jnp.arange(P, dtype=jnp.int32) % T).at[dest].set(tok)
    row_w = jnp.zeros((P,), jnp.float32).at[dest].set(w)

    nb_used = blk_end[E - 1]
    bidx = jnp.arange(NB, dtype=jnp.int32)
    be = jnp.searchsorted(blk_end, bidx, side="right").astype(jnp.int32)
    be_last = be[nb_used - 1]
    block_expert = jnp.where(bidx < nb_used, be, be_last)
    block_valid = (bidx < nb_used).astype(jnp.int32)
    return dest, gather_idx, row_w, block_expert, block_valid


def _make_sc_gather(n_rows, d, chunk):
    per_w = n_rows // NWORKERS
    n_chunks = per_w // chunk
    mesh = plsc.VectorSubcoreMesh(core_axis_name="c", subcore_axis_name="s")

    def body(src_hbm, idx_hbm, out_hbm, idx_v, rows_v, sem):
        wid = lax.axis_index("s") * 2 + lax.axis_index("c")
        base = wid * per_w

        def step(i, carry):
            off = base + i * chunk
            pltpu.sync_copy(idx_hbm.at[pl.ds(off, chunk)], idx_v)
            pltpu.async_copy(src_hbm.at[idx_v], rows_v, sem).wait()
            pltpu.sync_copy(rows_v, out_hbm.at[pl.ds(off, chunk)])
            return carry

        lax.fori_loop(0, n_chunks, step, 0)

    return pl.kernel(
        body,
        out_type=jax.ShapeDtypeStruct((n_rows, d), jnp.float32),
        mesh=mesh,
        scratch_types=[
            pltpu.VMEM((chunk,), jnp.int32),
            pltpu.VMEM((chunk, d), jnp.float32),
            pltpu.SemaphoreType.DMA,
        ],
    )


@functools.cache
def _sc_gather(n_rows, d, chunk):
    return _make_sc_gather(n_rows, d, chunk)


def _sc_gather_x(src, idx):
    return _sc_gather(P, D, 64)(src, idx)


def _sc_gather_y(src, idx):
    return _sc_gather(NE, O, 64)(src, idx)


def _grouped_body(be_ref, valid_ref, xs_ref, W1_ref, b1_ref, W2_ref, b2_ref,
                  rw_ref, out_ref):
    b = pl.program_id(0)

    @pl.when(valid_ref[b] == 1)
    def _():
        h = jnp.maximum(
            jnp.dot(xs_ref[...], W1_ref[0],
                    preferred_element_type=jnp.float32) + b1_ref[0], 0.0)
        y = jnp.dot(h, W2_ref[0], preferred_element_type=jnp.float32)
        out_ref[...] = (y + b2_ref[0]) * rw_ref[...]


def _grouped_mlp(xs, W1, b1, W2, b2, row_w, block_expert, block_valid):
    grid_spec = pltpu.PrefetchScalarGridSpec(
        num_scalar_prefetch=2,
        grid=(NB,),
        in_specs=[
            pl.BlockSpec((BT, D), lambda b, be, v: (b, 0)),
            pl.BlockSpec((1, D, H), lambda b, be, v: (be[b], 0, 0)),
            pl.BlockSpec((1, 1, H), lambda b, be, v: (be[b], 0, 0)),
            pl.BlockSpec((1, H, O), lambda b, be, v: (be[b], 0, 0)),
            pl.BlockSpec((1, 1, O), lambda b, be, v: (be[b], 0, 0)),
            pl.BlockSpec((BT, 1), lambda b, be, v: (b, 0)),
        ],
        out_specs=pl.BlockSpec((BT, O), lambda b, be, v: (b, 0)),
    )
    return pl.pallas_call(
        _grouped_body,
        grid_spec=grid_spec,
        out_shape=jax.ShapeDtypeStruct((P, O), jnp.float32),
    )(block_expert, block_valid, xs, W1, b1.reshape(E, 1, H), W2,
      b2.reshape(E, 1, O), row_w.reshape(P, 1))


def _combine_body(g_ref, out_ref):
    out_ref[...] = g_ref[:, 0, :] + g_ref[:, 1, :]


def _combine(g):
    btc = 512
    return pl.pallas_call(
        _combine_body,
        grid=(T // btc,),
        in_specs=[pl.BlockSpec((btc, K, O), lambda t: (t, 0, 0))],
        out_specs=pl.BlockSpec((btc, O), lambda t: (t, 0)),
        out_shape=jax.ShapeDtypeStruct((T, O), jnp.float32),
    )(g)


@jax.jit
def kernel(x, Wg, bg, W1, b1, W2, b2):
    top_i, top_w = _gating(x, Wg, bg)
    dest, gather_idx, row_w, block_expert, block_valid = _route_meta(
        top_i, top_w)
    xs = _sc_gather_x(x, gather_idx)
    yw = _grouped_mlp(xs, W1, b1, W2, b2, row_w, block_expert, block_valid)
    g = _sc_gather_y(yw, dest)
    return _combine(g.reshape(T, K, O))


# speedup vs baseline: 2.0434x; 1.0382x over previous
import functools

import jax
import jax.numpy as jnp
from jax import lax
from jax.experimental import pallas as pl
from jax.experimental.pallas import tpu as pltpu
from jax.experimental.pallas import tpu_sc as plsc

T = 2048
D = 1024
H = 2048
O = 1024
E = 8
K = 2

BT = 512
NE = T * K
NB = (NE + E * (BT - 1) + BT - 1) // BT
P = NB * BT

NWORKERS = 32


def _gating_body(x_ref, Wg_ref, bg_ref, ti_ref, tw_ref):
    logits = jnp.dot(x_ref[...], Wg_ref[...],
                     preferred_element_type=jnp.float32) + bg_ref[...]
    idx = lax.broadcasted_iota(jnp.int32, (T, E), 1)
    m1 = jnp.max(logits, axis=1, keepdims=True)
    i1 = jnp.min(jnp.where(logits >= m1, idx, E), axis=1, keepdims=True)
    rest = jnp.where(idx == i1, -jnp.inf, logits)
    m2 = jnp.max(rest, axis=1, keepdims=True)
    i2 = jnp.min(jnp.where(rest >= m2, idx, E), axis=1, keepdims=True)
    r = jnp.exp(m2 - m1)
    w1 = 1.0 / (1.0 + r)
    ti_ref[...] = jnp.concatenate([i1, i2], axis=1)
    tw_ref[...] = jnp.concatenate([w1, 1.0 - w1], axis=1)


def _gating(x, Wg, bg):
    return pl.pallas_call(
        _gating_body,
        out_shape=(jax.ShapeDtypeStruct((T, K), jnp.int32),
                   jax.ShapeDtypeStruct((T, K), jnp.float32)),
    )(x, Wg, bg.reshape(1, E))


def _route_meta(top_i, top_w):
    f = top_i.reshape(-1)
    w = top_w.reshape(-1)
    oh = (f[:, None] == jnp.arange(E, dtype=jnp.int32)[None, :])
    ohi = oh.astype(jnp.int32)
    ranks = jnp.cumsum(ohi, axis=0) - ohi
    rank = jnp.sum(ranks * ohi, axis=1)
    counts = jnp.sum(ohi, axis=0)
    nblk = (counts + BT - 1) // BT
    blk_end = jnp.cumsum(nblk)
    blk_start = jnp.concatenate([jnp.zeros((1,), jnp.int32), blk_end[:-1]])
    dest = (blk_start[f] * BT + rank).astype(jnp.int32)

    tok = (jnp.arange(NE, dtype=jnp.int32) // K)
    gather_idx = (jnp.arange(P, dtype=jnp.int32) % T).at[dest].set(tok)

    nb_used = blk_end[E - 1]
    bidx = jnp.arange(NB, dtype=jnp.int32)
    be = jnp.searchsorted(blk_end, bidx, side="right").astype(jnp.int32)
    be_last = be[nb_used - 1]
    block_expert = jnp.where(bidx < nb_used, be, be_last)
    block_valid = (bidx < nb_used).astype(jnp.int32)
    return dest, gather_idx, block_expert, block_valid


def _make_sc_gather(n_rows, d, chunk):
    per_w = n_rows // NWORKERS
    n_chunks = per_w // chunk
    mesh = plsc.VectorSubcoreMesh(core_axis_name="c", subcore_axis_name="s")

    def body(src_hbm, idx_hbm, out_hbm, idx_v, rows_v, sem):
        wid = lax.axis_index("s") * 2 + lax.axis_index("c")
        base = wid * per_w

        def step(i, carry):
            off = base + i * chunk
            pltpu.sync_copy(idx_hbm.at[pl.ds(off, chunk)], idx_v)
            pltpu.async_copy(src_hbm.at[idx_v], rows_v, sem).wait()
            pltpu.sync_copy(rows_v, out_hbm.at[pl.ds(off, chunk)])
            return carry

        lax.fori_loop(0, n_chunks, step, 0)

    return pl.kernel(
        body,
        out_type=jax.ShapeDtypeStruct((n_rows, d), jnp.float32),
        mesh=mesh,
        scratch_types=[
            pltpu.VMEM((chunk,), jnp.int32),
            pltpu.VMEM((chunk, d), jnp.float32),
            pltpu.SemaphoreType.DMA,
        ],
    )


@functools.cache
def _sc_gather(n_rows, d, chunk):
    return _make_sc_gather(n_rows, d, chunk)


def _sc_gather_x(src, idx):
    return _sc_gather(P, D, 64)(src, idx)


def _sc_gather_y(src, idx):
    return _sc_gather(NE, O, 64)(src, idx)


def _grouped_body(be_ref, valid_ref, xs_ref, W1_ref, b1_ref, W2_ref, b2_ref,
                  out_ref):
    b = pl.program_id(0)

    @pl.when(valid_ref[b] == 1)
    def _():
        h = jnp.maximum(
            jnp.dot(xs_ref[...], W1_ref[0],
                    preferred_element_type=jnp.float32) + b1_ref[0], 0.0)
        y = jnp.dot(h, W2_ref[0], preferred_element_type=jnp.float32)
        out_ref[...] = y + b2_ref[0]


def _grouped_mlp(xs, W1, b1, W2, b2, block_expert, block_valid):
    grid_spec = pltpu.PrefetchScalarGridSpec(
        num_scalar_prefetch=2,
        grid=(NB,),
        in_specs=[
            pl.BlockSpec((BT, D), lambda b, be, v: (b, 0)),
            pl.BlockSpec((1, D, H), lambda b, be, v: (be[b], 0, 0)),
            pl.BlockSpec((1, 1, H), lambda b, be, v: (be[b], 0, 0)),
            pl.BlockSpec((1, H, O), lambda b, be, v: (be[b], 0, 0)),
            pl.BlockSpec((1, 1, O), lambda b, be, v: (be[b], 0, 0)),
        ],
        out_specs=pl.BlockSpec((BT, O), lambda b, be, v: (b, 0)),
    )
    return pl.pallas_call(
        _grouped_body,
        grid_spec=grid_spec,
        out_shape=jax.ShapeDtypeStruct((P, O), jnp.float32),
    )(block_expert, block_valid, xs, W1, b1.reshape(E, 1, H), W2,
      b2.reshape(E, 1, O))


def _combine_body(g_ref, tw_ref, out_ref):
    w0 = tw_ref[:, 0:1]
    w1 = tw_ref[:, 1:2]
    out_ref[...] = w0 * g_ref[:, 0, :] + w1 * g_ref[:, 1, :]


def _combine(g, top_w):
    btc = 512
    return pl.pallas_call(
        _combine_body,
        grid=(T // btc,),
        in_specs=[pl.BlockSpec((btc, K, O), lambda t: (t, 0, 0)),
                  pl.BlockSpec((btc, K), lambda t: (t, 0))],
        out_specs=pl.BlockSpec((btc, O), lambda t: (t, 0)),
        out_shape=jax.ShapeDtypeStruct((T, O), jnp.float32),
    )(g, top_w)


@jax.jit
def kernel(x, Wg, bg, W1, b1, W2, b2):
    top_i, top_w = _gating(x, Wg, bg)
    dest, gather_idx, block_expert, block_valid = _route_meta(top_i, top_w)
    xs = _sc_gather_x(x, gather_idx)
    ys = _grouped_mlp(xs, W1, b1, W2, b2, block_expert, block_valid)
    g = _sc_gather_y(ys, dest)
    return _combine(g.reshape(T, K, O), top_w)


# speedup vs baseline: 2.7152x; 1.3288x over previous
import functools

import jax
import jax.numpy as jnp
from jax import lax
from jax.experimental import pallas as pl
from jax.experimental.pallas import tpu as pltpu
from jax.experimental.pallas import tpu_sc as plsc

T = 2048
D = 1024
H = 2048
O = 1024
E = 8
K = 2

BT = 512
NE = T * K
NB = (NE + E * (BT - 1) + BT - 1) // BT
P = NB * BT

NWORKERS = 32


def _gating_body(x_ref, Wg_ref, bg_ref, ti_ref, tw_ref):
    logits = jnp.dot(x_ref[...], Wg_ref[...],
                     preferred_element_type=jnp.float32) + bg_ref[...]
    idx = lax.broadcasted_iota(jnp.int32, (T, E), 1)
    m1 = jnp.max(logits, axis=1, keepdims=True)
    i1 = jnp.min(jnp.where(logits >= m1, idx, E), axis=1, keepdims=True)
    rest = jnp.where(idx == i1, -jnp.inf, logits)
    m2 = jnp.max(rest, axis=1, keepdims=True)
    i2 = jnp.min(jnp.where(rest >= m2, idx, E), axis=1, keepdims=True)
    r = jnp.exp(m2 - m1)
    w1 = 1.0 / (1.0 + r)
    ti_ref[...] = jnp.concatenate([i1, i2], axis=1)
    tw_ref[...] = jnp.concatenate([w1, 1.0 - w1], axis=1)


def _gating(x, Wg, bg):
    return pl.pallas_call(
        _gating_body,
        out_shape=(jax.ShapeDtypeStruct((T, K), jnp.int32),
                   jax.ShapeDtypeStruct((T, K), jnp.float32)),
    )(x, Wg, bg.reshape(1, E))


def _route_meta(top_i):
    f = top_i.T.reshape(-1)
    oh = (f[:, None] == jnp.arange(E, dtype=jnp.int32)[None, :])
    ohi = oh.astype(jnp.int32)
    ranks = jnp.cumsum(ohi, axis=0) - ohi
    rank = jnp.sum(ranks * ohi, axis=1)
    counts = jnp.sum(ohi, axis=0)
    nblk = (counts + BT - 1) // BT
    blk_end = jnp.cumsum(nblk)
    blk_start = jnp.concatenate([jnp.zeros((1,), jnp.int32), blk_end[:-1]])
    dest = (blk_start[f] * BT + rank).astype(jnp.int32)

    nb_used = blk_end[E - 1]
    bidx = jnp.arange(NB, dtype=jnp.int32)
    be = jnp.searchsorted(blk_end, bidx, side="right").astype(jnp.int32)
    be_last = be[nb_used - 1]
    block_expert = jnp.where(bidx < nb_used, be, be_last)
    block_valid = (bidx < nb_used).astype(jnp.int32)
    return dest, block_expert, block_valid


def _make_sc_gather(n_rows, d, chunk):
    per_w = n_rows // NWORKERS
    n_chunks = per_w // chunk
    mesh = plsc.VectorSubcoreMesh(core_axis_name="c", subcore_axis_name="s")

    def body(src_hbm, idx_hbm, out_hbm, idx_v, rows_v, sem):
        wid = lax.axis_index("s") * 2 + lax.axis_index("c")
        base = wid * per_w

        def step(i, carry):
            off = base + i * chunk
            pltpu.sync_copy(idx_hbm.at[pl.ds(off, chunk)], idx_v)
            pltpu.async_copy(src_hbm.at[idx_v], rows_v, sem).wait()
            pltpu.sync_copy(rows_v, out_hbm.at[pl.ds(off, chunk)])
            return carry

        lax.fori_loop(0, n_chunks, step, 0)

    return pl.kernel(
        body,
        out_type=jax.ShapeDtypeStruct((n_rows, d), jnp.float32),
        mesh=mesh,
        scratch_types=[
            pltpu.VMEM((chunk,), jnp.int32),
            pltpu.VMEM((chunk, d), jnp.float32),
            pltpu.SemaphoreType.DMA,
        ],
    )


@functools.cache
def _sc_gather(n_rows, d, chunk):
    return _make_sc_gather(n_rows, d, chunk)


def _sc_gather_y(src, idx):
    return _sc_gather(NE, O, 64)(src, idx)


def _make_sc_dispatch(chunk):
    per_w = NE // NWORKERS
    n_chunks = per_w // chunk
    mesh = plsc.VectorSubcoreMesh(core_axis_name="c", subcore_axis_name="s")

    def body(x_hbm, dest_hbm, tok_hbm, xs_hbm, tok_v, dest_v, rows_v, sem):
        wid = lax.axis_index("s") * 2 + lax.axis_index("c")
        base = wid * per_w

        def step(i, carry):
            off = base + i * chunk
            pltpu.sync_copy(tok_hbm.at[pl.ds(off, chunk)], tok_v)
            pltpu.sync_copy(dest_hbm.at[pl.ds(off, chunk)], dest_v)
            pltpu.async_copy(x_hbm.at[tok_v], rows_v, sem).wait()
            pltpu.async_copy(rows_v, xs_hbm.at[dest_v], sem).wait()
            return carry

        lax.fori_loop(0, n_chunks, step, 0)

    return pl.kernel(
        body,
        out_type=jax.ShapeDtypeStruct((P, D), jnp.float32),
        mesh=mesh,
        scratch_types=[
            pltpu.VMEM((chunk,), jnp.int32),
            pltpu.VMEM((chunk,), jnp.int32),
            pltpu.VMEM((chunk, D), jnp.float32),
            pltpu.SemaphoreType.DMA,
        ],
    )


@functools.cache
def _sc_dispatch(chunk):
    return _make_sc_dispatch(chunk)


def _sc_dispatch_x(x, dest):
    tok = jnp.tile(jnp.arange(T, dtype=jnp.int32), K)
    return _sc_dispatch(64)(x, dest, tok)


def _grouped_body(be_ref, valid_ref, xs_ref, W1_ref, b1_ref, W2_ref, b2_ref,
                  out_ref):
    b = pl.program_id(0)

    @pl.when(valid_ref[b] == 1)
    def _():
        h = jnp.maximum(
            jnp.dot(xs_ref[...], W1_ref[0],
                    preferred_element_type=jnp.float32) + b1_ref[0], 0.0)
        y = jnp.dot(h, W2_ref[0], preferred_element_type=jnp.float32)
        out_ref[...] = y + b2_ref[0]


def _grouped_mlp(xs, W1, b1, W2, b2, block_expert, block_valid):
    grid_spec = pltpu.PrefetchScalarGridSpec(
        num_scalar_prefetch=2,
        grid=(NB,),
        in_specs=[
            pl.BlockSpec((BT, D), lambda b, be, v: (b, 0)),
            pl.BlockSpec((1, D, H), lambda b, be, v: (be[b], 0, 0)),
            pl.BlockSpec((1, 1, H), lambda b, be, v: (be[b], 0, 0)),
            pl.BlockSpec((1, H, O), lambda b, be, v: (be[b], 0, 0)),
            pl.BlockSpec((1, 1, O), lambda b, be, v: (be[b], 0, 0)),
        ],
        out_specs=pl.BlockSpec((BT, O), lambda b, be, v: (b, 0)),
    )
    return pl.pallas_call(
        _grouped_body,
        grid_spec=grid_spec,
        out_shape=jax.ShapeDtypeStruct((P, O), jnp.float32),
    )(block_expert, block_valid, xs, W1, b1.reshape(E, 1, H), W2,
      b2.reshape(E, 1, O))


def _combine_body(g0_ref, g1_ref, tw_ref, out_ref):
    w0 = tw_ref[:, 0:1]
    w1 = tw_ref[:, 1:2]
    out_ref[...] = w0 * g0_ref[...] + w1 * g1_ref[...]


def _combine(g, top_w):
    btc = 512
    nblk = T // btc
    return pl.pallas_call(
        _combine_body,
        grid=(nblk,),
        in_specs=[pl.BlockSpec((btc, O), lambda t: (t, 0)),
                  pl.BlockSpec((btc, O), lambda t, n=nblk: (t + n, 0)),
                  pl.BlockSpec((btc, K), lambda t: (t, 0))],
        out_specs=pl.BlockSpec((btc, O), lambda t: (t, 0)),
        out_shape=jax.ShapeDtypeStruct((T, O), jnp.float32),
    )(g, g, top_w)


@jax.jit
def kernel(x, Wg, bg, W1, b1, W2, b2):
    top_i, top_w = _gating(x, Wg, bg)
    dest, block_expert, block_valid = _route_meta(top_i)
    xs = _sc_dispatch_x(x, dest)
    ys = _grouped_mlp(xs, W1, b1, W2, b2, block_expert, block_valid)
    g = _sc_gather_y(ys, dest)
    return _combine(g, top_w)
